# Initial kernel scaffold; baseline (speedup 1.0000x reference)
import jax
import jax.numpy as jnp
from jax.experimental import pallas as pl


def kernel(x_src, src_idx, dst_idx, W, b):
    raise NotImplementedError("write your pallas kernel here")


# speedup vs baseline: 104.4878x; 104.4878x over previous
import jax
import jax.numpy as jnp
from jax import lax
from jax.experimental import pallas as pl
from jax.experimental.pallas import tpu as pltpu
from jax.experimental.pallas import tpu_sc as plsc

V, T, NSRC, NDST, E, H = 2, 4, 50000, 12288, 800000, 128
P = V * T
NC, NS, L = 2, 16, 16
ROW = 128
RPT = 392
NB = 56
NBLK = RPT // NB
ROWS = NS * RPT
EPAD = ROWS * ROW
NSRC_PAD = 50176
SRC_CH = NSRC_PAD // NS
NDST_PAD = 12416
DST_CH = NDST_PAD // NS
PPC = P // NC
BN = 512


def _sc_body(feat_h, sidx_h, didx_h, agg_h, degin_h,
             degout_sp, val_sp, agg_sp, degin_sp,
             sidx_v, didx_v, eval_v, ones_v, fbuf, dbuf, zbuf):
    c = lax.axis_index("c")
    w = lax.axis_index("s")

    @pl.loop(0, SRC_CH // L)
    def _init_zeros(i):
        zbuf[pl.ds(i * L, L)] = jnp.zeros((L,), jnp.float32)

    @pl.loop(0, ROW // L)
    def _init_ones(i):
        ones_v[pl.ds(i * L, L)] = jnp.ones((L,), jnp.float32)

    @pl.loop(0, PPC)
    def _pair(k):
        p = c * PPC + k

        pltpu.sync_copy(zbuf, degout_sp.at[pl.ds(w * SRC_CH, SRC_CH)])
        pltpu.sync_copy(zbuf.at[pl.ds(0, DST_CH)],
                        agg_sp.at[pl.ds(w * DST_CH, DST_CH)])
        pltpu.sync_copy(zbuf.at[pl.ds(0, DST_CH)],
                        degin_sp.at[pl.ds(w * DST_CH, DST_CH)])
        plsc.subcore_barrier()

        @pl.loop(0, NBLK)
        def _blk1(bi):
            row0 = pl.multiple_of(w * RPT + bi * NB, 8)
            pltpu.sync_copy(sidx_h.at[p, pl.ds(row0, NB)], sidx_v)

            @pl.loop(0, NB)
            def _row1(j):
                pltpu.sync_copy(ones_v, degout_sp.at[sidx_v.at[j]], add=True)

        plsc.subcore_barrier()

        base = w * SRC_CH
        fbase = pl.multiple_of(p * NSRC_PAD + base, 8)
        pltpu.sync_copy(feat_h.at[pl.ds(fbase, SRC_CH)], fbuf)
        pltpu.sync_copy(degout_sp.at[pl.ds(base, SRC_CH)], dbuf)

        @pl.loop(0, SRC_CH // L)
        def _vec(i):
            off = i * L
            dg = jnp.maximum(dbuf[pl.ds(off, L)], 1.0)
            bits = lax.bitcast_convert_type(dg, jnp.int32)
            bits = jnp.int32(0x5F3759DF) - (bits >> 1)
            y = lax.bitcast_convert_type(bits, jnp.float32)
            hx = dg * 0.5
            y = y * (1.5 - hx * y * y)
            y = y * (1.5 - hx * y * y)
            y = y * (1.5 - hx * y * y)
            ft = fbuf[pl.ds(off, L)]
            ft = jnp.where(ft != ft, jnp.float32(0.0), ft)
            fbuf[pl.ds(off, L)] = ft * y

        pltpu.sync_copy(fbuf, val_sp.at[pl.ds(base, SRC_CH)])
        plsc.subcore_barrier()

        @pl.loop(0, NBLK)
        def _blk3(bi):
            row0 = pl.multiple_of(w * RPT + bi * NB, 8)
            pltpu.sync_copy(sidx_h.at[p, pl.ds(row0, NB)], sidx_v)
            pltpu.sync_copy(didx_h.at[p, pl.ds(row0, NB)], didx_v)

            @pl.loop(0, NB)
            def _row3(j):
                pltpu.sync_copy(val_sp.at[sidx_v.at[j]], eval_v.at[j])
                pltpu.sync_copy(eval_v.at[j], agg_sp.at[didx_v.at[j]],
                                add=True)
                pltpu.sync_copy(ones_v, degin_sp.at[didx_v.at[j]], add=True)

        plsc.subcore_barrier()

        off = w * DST_CH
        hoff = pl.multiple_of(p * NDST_PAD + off, 8)
        pltpu.sync_copy(agg_sp.at[pl.ds(off, DST_CH)],
                        fbuf.at[pl.ds(0, DST_CH)])
        pltpu.sync_copy(fbuf.at[pl.ds(0, DST_CH)],
                        agg_h.at[pl.ds(hoff, DST_CH)])
        pltpu.sync_copy(degin_sp.at[pl.ds(off, DST_CH)],
                        dbuf.at[pl.ds(0, DST_CH)])
        pltpu.sync_copy(dbuf.at[pl.ds(0, DST_CH)],
                        degin_h.at[pl.ds(hoff, DST_CH)])


def _tc_body(agg_r, deg_r, w_r, b_r, o_r):
    zall = agg_r[...] * lax.rsqrt(jnp.maximum(deg_r[...], 1.0))
    for p in range(P):
        x = zall[:, p:p + 1] * w_r[p:p + 1, :] + b_r[p:p + 1, :]
        o_r[:, p // T, p % T, :] = jnp.where(x >= 0, x, 0.01 * x)


def kernel(x_src, src_idx, dst_idx, W, b):
    feat = jnp.pad(x_src.reshape(P, NSRC), ((0, 0), (0, NSRC_PAD - NSRC)))
    npad = EPAD - E
    pad_s = NSRC + (jnp.arange(npad, dtype=jnp.int32) % (NSRC_PAD - NSRC))
    pad_d = NDST + (jnp.arange(npad, dtype=jnp.int32) % (NDST_PAD - NDST))
    sidx = jnp.concatenate(
        [src_idx.reshape(P, E), jnp.broadcast_to(pad_s, (P, npad))],
        axis=1).reshape(P, ROWS, ROW)
    didx = jnp.concatenate(
        [dst_idx.reshape(P, E), jnp.broadcast_to(pad_d, (P, npad))],
        axis=1).reshape(P, ROWS, ROW)

    sc = pl.kernel(
        _sc_body,
        out_type=(jax.ShapeDtypeStruct((P * NDST_PAD,), jnp.float32),
                  jax.ShapeDtypeStruct((P * NDST_PAD,), jnp.float32)),
        mesh=plsc.VectorSubcoreMesh(core_axis_name="c", subcore_axis_name="s"),
        scratch_types=[
            pltpu.VMEM_SHARED((NSRC_PAD,), jnp.float32),
            pltpu.VMEM_SHARED((NSRC_PAD,), jnp.float32),
            pltpu.VMEM_SHARED((NDST_PAD,), jnp.float32),
            pltpu.VMEM_SHARED((NDST_PAD,), jnp.float32),
            pltpu.VMEM((NB, ROW), jnp.int32),
            pltpu.VMEM((NB, ROW), jnp.int32),
            pltpu.VMEM((NB, ROW), jnp.float32),
            pltpu.VMEM((ROW,), jnp.float32),
            pltpu.VMEM((SRC_CH,), jnp.float32),
            pltpu.VMEM((SRC_CH,), jnp.float32),
            pltpu.VMEM((SRC_CH,), jnp.float32),
        ],
    )
    agg, degin = sc(feat.reshape(-1), sidx, didx)

    agg_t = agg.reshape(P, NDST_PAD)[:, :NDST].T
    deg_t = degin.reshape(P, NDST_PAD)[:, :NDST].T
    return pl.pallas_call(
        _tc_body,
        grid=(NDST // BN,),
        in_specs=[
            pl.BlockSpec((BN, P), lambda i: (i, 0)),
            pl.BlockSpec((BN, P), lambda i: (i, 0)),
            pl.BlockSpec((P, H), lambda i: (0, 0)),
            pl.BlockSpec((P, H), lambda i: (0, 0)),
        ],
        out_specs=pl.BlockSpec((BN, V, T, H), lambda i: (i, 0, 0, 0)),
        out_shape=jax.ShapeDtypeStruct((NDST, V, T, H), jnp.float32),
    )(agg_t, deg_t, W.reshape(P, H), b.reshape(P, H))


# speedup vs baseline: 194.4308x; 1.8608x over previous
import jax
import jax.numpy as jnp
from jax import lax
from jax.experimental import pallas as pl
from jax.experimental.pallas import tpu as pltpu
from jax.experimental.pallas import tpu_sc as plsc

V, T, NSRC, NDST, E, H = 2, 4, 50000, 12288, 800000, 128
P = V * T
NC, NS, L = 2, 16, 16
ROW = 128
RPT = 392
NB = 56
NBLK = RPT // NB
ROWS = NS * RPT
EPAD = ROWS * ROW
NSRC_PAD = 50176
SRC_CH = NSRC_PAD // NS
NDST_PAD = 12416
DST_CH = NDST_PAD // NS
PPC = P // NC
BN = 512


def _sc_body(feat_h, sidx_h, didx_h, agg_h, degin_h,
             degout_sp, val_sp, agg_sp, degin_sp,
             sidx_v, didx_v, eval_v, ones_v, fbuf, dbuf, zbuf,
             sem0, sem1, sem2):
    c = lax.axis_index("c")
    w = lax.axis_index("s")

    @pl.loop(0, SRC_CH // L)
    def _init_zeros(i):
        zbuf[pl.ds(i * L, L)] = jnp.zeros((L,), jnp.float32)

    @pl.loop(0, ROW // L)
    def _init_ones(i):
        ones_v[pl.ds(i * L, L)] = jnp.ones((L,), jnp.float32)

    @pl.loop(0, PPC)
    def _pair(k):
        p = c * PPC + k

        pltpu.sync_copy(zbuf, degout_sp.at[pl.ds(w * SRC_CH, SRC_CH)])
        pltpu.sync_copy(zbuf.at[pl.ds(0, DST_CH)],
                        agg_sp.at[pl.ds(w * DST_CH, DST_CH)])
        pltpu.sync_copy(zbuf.at[pl.ds(0, DST_CH)],
                        degin_sp.at[pl.ds(w * DST_CH, DST_CH)])
        plsc.subcore_barrier()

        @pl.loop(0, NBLK)
        def _blk1(bi):
            row0 = pl.multiple_of(w * RPT + bi * NB, 8)
            pltpu.sync_copy(sidx_h.at[p, pl.ds(row0, NB)], sidx_v)
            hs = [pltpu.async_copy(ones_v, degout_sp.at[sidx_v.at[j]],
                                   sem0, add=True) for j in range(NB)]
            for d in hs:
                d.wait()

        plsc.subcore_barrier()

        base = w * SRC_CH
        fbase = pl.multiple_of(p * NSRC_PAD + base, 8)
        pltpu.sync_copy(feat_h.at[pl.ds(fbase, SRC_CH)], fbuf)
        pltpu.sync_copy(degout_sp.at[pl.ds(base, SRC_CH)], dbuf)

        @pl.loop(0, SRC_CH // L)
        def _vec(i):
            off = i * L
            dg = jnp.maximum(dbuf[pl.ds(off, L)], 1.0)
            bits = lax.bitcast_convert_type(dg, jnp.int32)
            bits = jnp.int32(0x5F3759DF) - (bits >> 1)
            y = lax.bitcast_convert_type(bits, jnp.float32)
            hx = dg * 0.5
            y = y * (1.5 - hx * y * y)
            y = y * (1.5 - hx * y * y)
            y = y * (1.5 - hx * y * y)
            ft = fbuf[pl.ds(off, L)]
            ft = jnp.where(ft != ft, jnp.float32(0.0), ft)
            fbuf[pl.ds(off, L)] = ft * y

        pltpu.sync_copy(fbuf, val_sp.at[pl.ds(base, SRC_CH)])
        plsc.subcore_barrier()

        @pl.loop(0, NBLK)
        def _blk3(bi):
            row0 = pl.multiple_of(w * RPT + bi * NB, 8)
            pltpu.sync_copy(sidx_h.at[p, pl.ds(row0, NB)], sidx_v)
            pltpu.sync_copy(didx_h.at[p, pl.ds(row0, NB)], didx_v)
            gs = [pltpu.async_copy(val_sp.at[sidx_v.at[j]], eval_v.at[j],
                                   sem0) for j in range(NB)]
            hs = [pltpu.async_copy(ones_v, degin_sp.at[didx_v.at[j]],
                                   sem1, add=True) for j in range(NB)]
            for d in gs:
                d.wait()
            ss = [pltpu.async_copy(eval_v.at[j], agg_sp.at[didx_v.at[j]],
                                   sem2, add=True) for j in range(NB)]
            for d in hs:
                d.wait()
            for d in ss:
                d.wait()

        plsc.subcore_barrier()

        off = w * DST_CH
        hoff = pl.multiple_of(p * NDST_PAD + off, 8)
        pltpu.sync_copy(agg_sp.at[pl.ds(off, DST_CH)],
                        fbuf.at[pl.ds(0, DST_CH)])
        pltpu.sync_copy(fbuf.at[pl.ds(0, DST_CH)],
                        agg_h.at[pl.ds(hoff, DST_CH)])
        pltpu.sync_copy(degin_sp.at[pl.ds(off, DST_CH)],
                        dbuf.at[pl.ds(0, DST_CH)])
        pltpu.sync_copy(dbuf.at[pl.ds(0, DST_CH)],
                        degin_h.at[pl.ds(hoff, DST_CH)])


def _tc_body(agg_r, deg_r, w_r, b_r, o_r):
    zall = agg_r[...] * lax.rsqrt(jnp.maximum(deg_r[...], 1.0))
    for p in range(P):
        x = zall[:, p:p + 1] * w_r[p:p + 1, :] + b_r[p:p + 1, :]
        o_r[:, p // T, p % T, :] = jnp.where(x >= 0, x, 0.01 * x)


def kernel(x_src, src_idx, dst_idx, W, b):
    feat = jnp.pad(x_src.reshape(P, NSRC), ((0, 0), (0, NSRC_PAD - NSRC)))
    npad = EPAD - E
    pad_s = NSRC + (jnp.arange(npad, dtype=jnp.int32) % (NSRC_PAD - NSRC))
    pad_d = NDST + (jnp.arange(npad, dtype=jnp.int32) % (NDST_PAD - NDST))
    sidx = jnp.concatenate(
        [src_idx.reshape(P, E), jnp.broadcast_to(pad_s, (P, npad))],
        axis=1).reshape(P, ROWS, ROW)
    didx = jnp.concatenate(
        [dst_idx.reshape(P, E), jnp.broadcast_to(pad_d, (P, npad))],
        axis=1).reshape(P, ROWS, ROW)

    sc = pl.kernel(
        _sc_body,
        out_type=(jax.ShapeDtypeStruct((P * NDST_PAD,), jnp.float32),
                  jax.ShapeDtypeStruct((P * NDST_PAD,), jnp.float32)),
        mesh=plsc.VectorSubcoreMesh(core_axis_name="c", subcore_axis_name="s"),
        scratch_types=[
            pltpu.VMEM_SHARED((NSRC_PAD,), jnp.float32),
            pltpu.VMEM_SHARED((NSRC_PAD,), jnp.float32),
            pltpu.VMEM_SHARED((NDST_PAD,), jnp.float32),
            pltpu.VMEM_SHARED((NDST_PAD,), jnp.float32),
            pltpu.VMEM((NB, ROW), jnp.int32),
            pltpu.VMEM((NB, ROW), jnp.int32),
            pltpu.VMEM((NB, ROW), jnp.float32),
            pltpu.VMEM((ROW,), jnp.float32),
            pltpu.VMEM((SRC_CH,), jnp.float32),
            pltpu.VMEM((SRC_CH,), jnp.float32),
            pltpu.VMEM((SRC_CH,), jnp.float32),
            pltpu.SemaphoreType.DMA,
            pltpu.SemaphoreType.DMA,
            pltpu.SemaphoreType.DMA,
        ],
    )
    agg, degin = sc(feat.reshape(-1), sidx, didx)

    agg_t = agg.reshape(P, NDST_PAD)[:, :NDST].T
    deg_t = degin.reshape(P, NDST_PAD)[:, :NDST].T
    return pl.pallas_call(
        _tc_body,
        grid=(NDST // BN,),
        in_specs=[
            pl.BlockSpec((BN, P), lambda i: (i, 0)),
            pl.BlockSpec((BN, P), lambda i: (i, 0)),
            pl.BlockSpec((P, H), lambda i: (0, 0)),
            pl.BlockSpec((P, H), lambda i: (0, 0)),
        ],
        out_specs=pl.BlockSpec((BN, V, T, H), lambda i: (i, 0, 0, 0)),
        out_shape=jax.ShapeDtypeStruct((NDST, V, T, H), jnp.float32),
    )(agg_t, deg_t, W.reshape(P, H), b.reshape(P, H))


# speedup vs baseline: 220.7402x; 1.1353x over previous
import jax
import jax.numpy as jnp
from jax import lax
from jax.experimental import pallas as pl
from jax.experimental.pallas import tpu as pltpu
from jax.experimental.pallas import tpu_sc as plsc

V, T, NSRC, NDST, E, H = 2, 4, 50000, 12288, 800000, 128
P = V * T
NC, NS, L = 2, 16, 16
ROW = 128
ROWS = E // ROW
RPT = 392
NB = 56
NBLK = RPT // NB
TAILB = 6
TAILR = ROWS - 15 * RPT - TAILB * NB
NSRC_PAD = 50176
SRC_CH = NSRC_PAD // NS
NDST_PAD = 12416
DST_CH = NDST_PAD // NS
PPC = P // NC
BN = 512


def _sc_body(feat_h, sidx_h, didx_h, agg_h, degin_h,
             degout_sp, val_sp, agg_sp, degin_sp,
             sidx_v, didx_v, eval_v, ones_v, fbuf, dbuf, zbuf,
             sem0, sem1, sem2):
    c = lax.axis_index("c")
    w = lax.axis_index("s")

    @pl.loop(0, SRC_CH // L)
    def _init_zeros(i):
        zbuf[pl.ds(i * L, L)] = jnp.zeros((L,), jnp.float32)

    @pl.loop(0, ROW // L)
    def _init_ones(i):
        ones_v[pl.ds(i * L, L)] = jnp.ones((L,), jnp.float32)

    @pl.loop(0, PPC)
    def _pair(k):
        p = c * PPC + k

        pltpu.sync_copy(zbuf, degout_sp.at[pl.ds(w * SRC_CH, SRC_CH)])
        pltpu.sync_copy(zbuf.at[pl.ds(0, DST_CH)],
                        agg_sp.at[pl.ds(w * DST_CH, DST_CH)])
        pltpu.sync_copy(zbuf.at[pl.ds(0, DST_CH)],
                        degin_sp.at[pl.ds(w * DST_CH, DST_CH)])
        plsc.subcore_barrier()

        def _hist_block(row0, nrows):
            pltpu.sync_copy(sidx_h.at[p, pl.ds(row0, nrows)],
                            sidx_v.at[pl.ds(0, nrows)])
            hs = [pltpu.async_copy(ones_v, degout_sp.at[sidx_v.at[j]],
                                   sem0, add=True) for j in range(nrows)]
            for d in hs:
                d.wait()

        nblk = jnp.where(w < NS - 1, NBLK, TAILB)

        @pl.loop(0, nblk)
        def _blk1(bi):
            _hist_block(pl.multiple_of(w * RPT + bi * NB, 8), NB)

        @pl.when(w == NS - 1)
        def _tail1():
            _hist_block((NS - 1) * RPT + TAILB * NB, TAILR)

        plsc.subcore_barrier()

        base = w * SRC_CH
        fbase = pl.multiple_of(p * NSRC_PAD + base, 8)
        pltpu.sync_copy(feat_h.at[pl.ds(fbase, SRC_CH)], fbuf)
        pltpu.sync_copy(degout_sp.at[pl.ds(base, SRC_CH)], dbuf)

        @pl.loop(0, SRC_CH // L)
        def _vec(i):
            off = i * L
            dg = jnp.maximum(dbuf[pl.ds(off, L)], 1.0)
            bits = lax.bitcast_convert_type(dg, jnp.int32)
            bits = jnp.int32(0x5F3759DF) - (bits >> 1)
            y = lax.bitcast_convert_type(bits, jnp.float32)
            hx = dg * 0.5
            y = y * (1.5 - hx * y * y)
            y = y * (1.5 - hx * y * y)
            y = y * (1.5 - hx * y * y)
            ft = fbuf[pl.ds(off, L)]
            ft = jnp.where(ft != ft, jnp.float32(0.0), ft)
            fbuf[pl.ds(off, L)] = ft * y

        pltpu.sync_copy(fbuf, val_sp.at[pl.ds(base, SRC_CH)])
        plsc.subcore_barrier()

        def _agg_block(row0, nrows):
            pltpu.sync_copy(sidx_h.at[p, pl.ds(row0, nrows)],
                            sidx_v.at[pl.ds(0, nrows)])
            pltpu.sync_copy(didx_h.at[p, pl.ds(row0, nrows)],
                            didx_v.at[pl.ds(0, nrows)])
            gs = [pltpu.async_copy(val_sp.at[sidx_v.at[j]], eval_v.at[j],
                                   sem0) for j in range(nrows)]
            hs = [pltpu.async_copy(ones_v, degin_sp.at[didx_v.at[j]],
                                   sem1, add=True) for j in range(nrows)]
            for d in gs:
                d.wait()
            ss = [pltpu.async_copy(eval_v.at[j], agg_sp.at[didx_v.at[j]],
                                   sem2, add=True) for j in range(nrows)]
            for d in hs:
                d.wait()
            for d in ss:
                d.wait()

        @pl.loop(0, nblk)
        def _blk3(bi):
            _agg_block(pl.multiple_of(w * RPT + bi * NB, 8), NB)

        @pl.when(w == NS - 1)
        def _tail3():
            _agg_block((NS - 1) * RPT + TAILB * NB, TAILR)

        plsc.subcore_barrier()

        off = w * DST_CH
        hoff = pl.multiple_of(p * NDST_PAD + off, 8)
        pltpu.sync_copy(agg_sp.at[pl.ds(off, DST_CH)],
                        fbuf.at[pl.ds(0, DST_CH)])
        pltpu.sync_copy(fbuf.at[pl.ds(0, DST_CH)],
                        agg_h.at[pl.ds(hoff, DST_CH)])
        pltpu.sync_copy(degin_sp.at[pl.ds(off, DST_CH)],
                        dbuf.at[pl.ds(0, DST_CH)])
        pltpu.sync_copy(dbuf.at[pl.ds(0, DST_CH)],
                        degin_h.at[pl.ds(hoff, DST_CH)])


def _tc_body(agg_r, deg_r, w_r, b_r, o_r):
    zall = agg_r[...] * lax.rsqrt(jnp.maximum(deg_r[...], 1.0))
    for p in range(P):
        x = zall[:, p:p + 1] * w_r[p:p + 1, :] + b_r[p:p + 1, :]
        o_r[:, p // T, p % T, :] = jnp.where(x >= 0, x, 0.01 * x)


def kernel(x_src, src_idx, dst_idx, W, b):
    feat = jnp.pad(x_src.reshape(P, NSRC), ((0, 0), (0, NSRC_PAD - NSRC)))
    sidx = src_idx.reshape(P, ROWS, ROW)
    didx = dst_idx.reshape(P, ROWS, ROW)

    sc = pl.kernel(
        _sc_body,
        out_type=(jax.ShapeDtypeStruct((P * NDST_PAD,), jnp.float32),
                  jax.ShapeDtypeStruct((P * NDST_PAD,), jnp.float32)),
        mesh=plsc.VectorSubcoreMesh(core_axis_name="c", subcore_axis_name="s"),
        scratch_types=[
            pltpu.VMEM_SHARED((NSRC_PAD,), jnp.float32),
            pltpu.VMEM_SHARED((NSRC_PAD,), jnp.float32),
            pltpu.VMEM_SHARED((NDST_PAD,), jnp.float32),
            pltpu.VMEM_SHARED((NDST_PAD,), jnp.float32),
            pltpu.VMEM((NB, ROW), jnp.int32),
            pltpu.VMEM((NB, ROW), jnp.int32),
            pltpu.VMEM((NB, ROW), jnp.float32),
            pltpu.VMEM((ROW,), jnp.float32),
            pltpu.VMEM((SRC_CH,), jnp.float32),
            pltpu.VMEM((SRC_CH,), jnp.float32),
            pltpu.VMEM((SRC_CH,), jnp.float32),
            pltpu.SemaphoreType.DMA,
            pltpu.SemaphoreType.DMA,
            pltpu.SemaphoreType.DMA,
        ],
    )
    agg, degin = sc(feat.reshape(-1), sidx, didx)

    agg_t = agg.reshape(P, NDST_PAD)[:, :NDST].T
    deg_t = degin.reshape(P, NDST_PAD)[:, :NDST].T
    return pl.pallas_call(
        _tc_body,
        grid=(NDST // BN,),
        in_specs=[
            pl.BlockSpec((BN, P), lambda i: (i, 0)),
            pl.BlockSpec((BN, P), lambda i: (i, 0)),
            pl.BlockSpec((P, H), lambda i: (0, 0)),
            pl.BlockSpec((P, H), lambda i: (0, 0)),
        ],
        out_specs=pl.BlockSpec((BN, V, T, H), lambda i: (i, 0, 0, 0)),
        out_shape=jax.ShapeDtypeStruct((NDST, V, T, H), jnp.float32),
    )(agg_t, deg_t, W.reshape(P, H), b.reshape(P, H))


# speedup vs baseline: 223.1366x; 1.0109x over previous
import jax
import jax.numpy as jnp
from jax import lax
from jax.experimental import pallas as pl
from jax.experimental.pallas import tpu as pltpu
from jax.experimental.pallas import tpu_sc as plsc

V, T, NSRC, NDST, E, H = 2, 4, 50000, 12288, 800000, 128
P = V * T
NC, NS, L = 2, 16, 16
ROW = 128
ROWS = E // ROW
RPT = 392
NB = 56
NBLK = RPT // NB
TAILB = 6
TAILR = ROWS - 15 * RPT - TAILB * NB
NSRC_PAD = 50176
SRC_CH = NSRC_PAD // NS
NDST_PAD = 12416
DST_CH = NDST_PAD // NS
PPC = P // NC
BN = 512


def _sc_body(feat_h, sidx_h, didx_h, agg_h, degin_h,
             degout_sp, val_sp, agg_sp, degin_sp,
             sidx_v, didx_v, eval_v, ones_v, fbuf, dbuf, zbuf,
             sem0, sem1, sem2):
    c = lax.axis_index("c")
    w = lax.axis_index("s")

    @pl.loop(0, SRC_CH // L)
    def _init_zeros(i):
        zbuf[pl.ds(i * L, L)] = jnp.zeros((L,), jnp.float32)

    @pl.loop(0, ROW // L)
    def _init_ones(i):
        ones_v[pl.ds(i * L, L)] = jnp.ones((L,), jnp.float32)

    @pl.loop(0, PPC)
    def _pair(k):
        p = c * PPC + k

        pltpu.sync_copy(zbuf, degout_sp.at[pl.ds(w * SRC_CH, SRC_CH)])
        pltpu.sync_copy(zbuf.at[pl.ds(0, DST_CH)],
                        agg_sp.at[pl.ds(w * DST_CH, DST_CH)])
        pltpu.sync_copy(zbuf.at[pl.ds(0, DST_CH)],
                        degin_sp.at[pl.ds(w * DST_CH, DST_CH)])
        plsc.subcore_barrier()

        def _hist_block(row0, nrows):
            pltpu.sync_copy(sidx_h.at[p, pl.ds(row0, nrows)],
                            sidx_v.at[pl.ds(0, nrows)])
            hs = [pltpu.async_copy(ones_v, degout_sp.at[sidx_v.at[j]],
                                   sem0, add=True) for j in range(nrows)]
            for d in hs:
                d.wait()

        nblk = jnp.where(w < NS - 1, NBLK, TAILB)

        @pl.loop(0, nblk)
        def _blk1(bi):
            _hist_block(pl.multiple_of(w * RPT + bi * NB, 8), NB)

        @pl.when(w == NS - 1)
        def _tail1():
            _hist_block((NS - 1) * RPT + TAILB * NB, TAILR)

        plsc.subcore_barrier()

        base = w * SRC_CH
        fbase = pl.multiple_of(p * NSRC_PAD + base, 8)
        pltpu.sync_copy(feat_h.at[pl.ds(fbase, SRC_CH)], fbuf)
        pltpu.sync_copy(degout_sp.at[pl.ds(base, SRC_CH)], dbuf)

        @pl.loop(0, SRC_CH // L)
        def _vec(i):
            off = i * L
            dg = jnp.maximum(dbuf[pl.ds(off, L)], 1.0)
            bits = lax.bitcast_convert_type(dg, jnp.int32)
            bits = jnp.int32(0x5F3759DF) - (bits >> 1)
            y = lax.bitcast_convert_type(bits, jnp.float32)
            hx = dg * 0.5
            y = y * (1.5 - hx * y * y)
            y = y * (1.5 - hx * y * y)
            y = y * (1.5 - hx * y * y)
            ft = fbuf[pl.ds(off, L)]
            ft = jnp.where(ft != ft, jnp.float32(0.0), ft)
            fbuf[pl.ds(off, L)] = ft * y

        pltpu.sync_copy(fbuf, val_sp.at[pl.ds(base, SRC_CH)])
        plsc.subcore_barrier()

        def _agg_block(row0, nrows):
            pltpu.sync_copy(sidx_h.at[p, pl.ds(row0, nrows)],
                            sidx_v.at[pl.ds(0, nrows)])
            pltpu.sync_copy(didx_h.at[p, pl.ds(row0, nrows)],
                            didx_v.at[pl.ds(0, nrows)])
            gs = [pltpu.async_copy(val_sp.at[sidx_v.at[j]], eval_v.at[j],
                                   sem0) for j in range(nrows)]
            hs = [pltpu.async_copy(ones_v, degin_sp.at[didx_v.at[j]],
                                   sem1, add=True) for j in range(nrows)]
            for d in gs:
                d.wait()
            ss = [pltpu.async_copy(eval_v.at[j], agg_sp.at[didx_v.at[j]],
                                   sem2, add=True) for j in range(nrows)]
            for d in hs:
                d.wait()
            for d in ss:
                d.wait()

        @pl.loop(0, nblk)
        def _blk3(bi):
            _agg_block(pl.multiple_of(w * RPT + bi * NB, 8), NB)

        @pl.when(w == NS - 1)
        def _tail3():
            _agg_block((NS - 1) * RPT + TAILB * NB, TAILR)

        plsc.subcore_barrier()

        off = w * DST_CH
        hoff = pl.multiple_of(p * NDST_PAD + off, 8)
        pltpu.sync_copy(agg_sp.at[pl.ds(off, DST_CH)],
                        fbuf.at[pl.ds(0, DST_CH)])
        pltpu.sync_copy(fbuf.at[pl.ds(0, DST_CH)],
                        agg_h.at[pl.ds(hoff, DST_CH)])
        pltpu.sync_copy(degin_sp.at[pl.ds(off, DST_CH)],
                        dbuf.at[pl.ds(0, DST_CH)])
        pltpu.sync_copy(dbuf.at[pl.ds(0, DST_CH)],
                        degin_h.at[pl.ds(hoff, DST_CH)])


def _tc_body(agg_r, deg_r, w_r, b_r, o_r):
    zall = (agg_r[...] * lax.rsqrt(jnp.maximum(deg_r[...], 1.0))).T
    for p in range(P):
        x = zall[:, p:p + 1] * w_r[p:p + 1, :] + b_r[p:p + 1, :]
        o_r[:, p // T, p % T, :] = jnp.where(x >= 0, x, 0.01 * x)


def kernel(x_src, src_idx, dst_idx, W, b):
    feat = jnp.pad(x_src.reshape(P, NSRC), ((0, 0), (0, NSRC_PAD - NSRC)))
    sidx = src_idx.reshape(P, ROWS, ROW)
    didx = dst_idx.reshape(P, ROWS, ROW)

    sc = pl.kernel(
        _sc_body,
        out_type=(jax.ShapeDtypeStruct((P * NDST_PAD,), jnp.float32),
                  jax.ShapeDtypeStruct((P * NDST_PAD,), jnp.float32)),
        mesh=plsc.VectorSubcoreMesh(core_axis_name="c", subcore_axis_name="s"),
        scratch_types=[
            pltpu.VMEM_SHARED((NSRC_PAD,), jnp.float32),
            pltpu.VMEM_SHARED((NSRC_PAD,), jnp.float32),
            pltpu.VMEM_SHARED((NDST_PAD,), jnp.float32),
            pltpu.VMEM_SHARED((NDST_PAD,), jnp.float32),
            pltpu.VMEM((NB, ROW), jnp.int32),
            pltpu.VMEM((NB, ROW), jnp.int32),
            pltpu.VMEM((NB, ROW), jnp.float32),
            pltpu.VMEM((ROW,), jnp.float32),
            pltpu.VMEM((SRC_CH,), jnp.float32),
            pltpu.VMEM((SRC_CH,), jnp.float32),
            pltpu.VMEM((SRC_CH,), jnp.float32),
            pltpu.SemaphoreType.DMA,
            pltpu.SemaphoreType.DMA,
            pltpu.SemaphoreType.DMA,
        ],
    )
    agg, degin = sc(feat.reshape(-1), sidx, didx)

    return pl.pallas_call(
        _tc_body,
        grid=(NDST // BN,),
        in_specs=[
            pl.BlockSpec((P, BN), lambda i: (0, i)),
            pl.BlockSpec((P, BN), lambda i: (0, i)),
            pl.BlockSpec((P, H), lambda i: (0, 0)),
            pl.BlockSpec((P, H), lambda i: (0, 0)),
        ],
        out_specs=pl.BlockSpec((BN, V, T, H), lambda i: (i, 0, 0, 0)),
        out_shape=jax.ShapeDtypeStruct((NDST, V, T, H), jnp.float32),
    )(agg.reshape(P, NDST_PAD), degin.reshape(P, NDST_PAD),
      W.reshape(P, H), b.reshape(P, H))


# speedup vs baseline: 243.0399x; 1.0892x over previous
import jax
import jax.numpy as jnp
from jax import lax
from jax.experimental import pallas as pl
from jax.experimental.pallas import tpu as pltpu
from jax.experimental.pallas import tpu_sc as plsc

V, T, NSRC, NDST, E, H = 2, 4, 50000, 12288, 800000, 128
P = V * T
NC, NS, L = 2, 16, 16
ROW = 128
ROWS = E // ROW
RPT = 392
NB = 56
NBLK = RPT // NB
TAILB = 6
TAILR = ROWS - 15 * RPT - TAILB * NB
NSRC_PAD = 50176
SRC_CH = NSRC_PAD // NS
NDST_PAD = 12416
DST_CH = NDST_PAD // NS
PPC = P // NC
BN = 512


def _sc_body(feat_h, sidx_h, didx_h, agg_h, degin_h,
             degout_sp, val_sp, agg_sp, degin_sp,
             sidx_v, didx_v, eval_v, ones_v, fbuf, dbuf, zbuf,
             sem0, sem1, sem2):
    c = lax.axis_index("c")
    w = lax.axis_index("s")

    @pl.loop(0, SRC_CH // L)
    def _init_zeros(i):
        zbuf[pl.ds(i * L, L)] = jnp.zeros((L,), jnp.float32)

    @pl.loop(0, ROW // L)
    def _init_ones(i):
        ones_v[pl.ds(i * L, L)] = jnp.ones((L,), jnp.float32)

    @pl.loop(0, PPC)
    def _pair(k):
        p = c * PPC + k

        pltpu.sync_copy(zbuf, degout_sp.at[pl.ds(w * SRC_CH, SRC_CH)])
        pltpu.sync_copy(zbuf.at[pl.ds(0, DST_CH)],
                        agg_sp.at[pl.ds(w * DST_CH, DST_CH)])
        pltpu.sync_copy(zbuf.at[pl.ds(0, DST_CH)],
                        degin_sp.at[pl.ds(w * DST_CH, DST_CH)])
        plsc.subcore_barrier()

        def _hist_block(row0, lrow0, nrows):
            pltpu.sync_copy(sidx_h.at[p, pl.ds(row0, nrows)],
                            sidx_v.at[pl.ds(lrow0, nrows)])
            dd = pltpu.async_copy(didx_h.at[p, pl.ds(row0, nrows)],
                                  didx_v.at[pl.ds(lrow0, nrows)], sem1)
            hs = [pltpu.async_copy(ones_v,
                                   degout_sp.at[sidx_v.at[lrow0 + j]],
                                   sem0, add=True) for j in range(nrows)]
            for d in hs:
                d.wait()
            dd.wait()

        nblk = jnp.where(w < NS - 1, NBLK, TAILB)

        @pl.loop(0, nblk)
        def _blk1(bi):
            _hist_block(pl.multiple_of(w * RPT + bi * NB, 8), bi * NB, NB)

        @pl.when(w == NS - 1)
        def _tail1():
            _hist_block((NS - 1) * RPT + TAILB * NB, TAILB * NB, TAILR)

        plsc.subcore_barrier()

        base = w * SRC_CH
        fbase = pl.multiple_of(p * NSRC_PAD + base, 8)
        pltpu.sync_copy(feat_h.at[pl.ds(fbase, SRC_CH)], fbuf)
        pltpu.sync_copy(degout_sp.at[pl.ds(base, SRC_CH)], dbuf)

        @pl.loop(0, SRC_CH // L)
        def _vec(i):
            off = i * L
            dg = jnp.maximum(dbuf[pl.ds(off, L)], 1.0)
            bits = lax.bitcast_convert_type(dg, jnp.int32)
            bits = jnp.int32(0x5F3759DF) - (bits >> 1)
            y = lax.bitcast_convert_type(bits, jnp.float32)
            hx = dg * 0.5
            y = y * (1.5 - hx * y * y)
            y = y * (1.5 - hx * y * y)
            y = y * (1.5 - hx * y * y)
            ft = fbuf[pl.ds(off, L)]
            ft = jnp.where(ft != ft, jnp.float32(0.0), ft)
            fbuf[pl.ds(off, L)] = ft * y

        pltpu.sync_copy(fbuf, val_sp.at[pl.ds(base, SRC_CH)])
        plsc.subcore_barrier()

        def _agg_block(lrow0, nrows):
            gs = [pltpu.async_copy(val_sp.at[sidx_v.at[lrow0 + j]],
                                   eval_v.at[j], sem0)
                  for j in range(nrows)]
            hs = [pltpu.async_copy(ones_v,
                                   degin_sp.at[didx_v.at[lrow0 + j]],
                                   sem1, add=True) for j in range(nrows)]
            for d in gs:
                d.wait()
            ss = [pltpu.async_copy(eval_v.at[j],
                                   agg_sp.at[didx_v.at[lrow0 + j]],
                                   sem2, add=True) for j in range(nrows)]
            for d in hs:
                d.wait()
            for d in ss:
                d.wait()

        @pl.loop(0, nblk)
        def _blk3(bi):
            _agg_block(bi * NB, NB)

        @pl.when(w == NS - 1)
        def _tail3():
            _agg_block(TAILB * NB, TAILR)

        plsc.subcore_barrier()

        off = w * DST_CH
        hoff = pl.multiple_of(p * NDST_PAD + off, 8)
        pltpu.sync_copy(agg_sp.at[pl.ds(off, DST_CH)],
                        fbuf.at[pl.ds(0, DST_CH)])
        pltpu.sync_copy(fbuf.at[pl.ds(0, DST_CH)],
                        agg_h.at[pl.ds(hoff, DST_CH)])
        pltpu.sync_copy(degin_sp.at[pl.ds(off, DST_CH)],
                        dbuf.at[pl.ds(0, DST_CH)])
        pltpu.sync_copy(dbuf.at[pl.ds(0, DST_CH)],
                        degin_h.at[pl.ds(hoff, DST_CH)])


def _tc_body(agg_r, deg_r, w_r, b_r, o_r):
    zall = (agg_r[...] * lax.rsqrt(jnp.maximum(deg_r[...], 1.0))).T
    for p in range(P):
        x = zall[:, p:p + 1] * w_r[p:p + 1, :] + b_r[p:p + 1, :]
        o_r[:, p // T, p % T, :] = jnp.where(x >= 0, x, 0.01 * x)


def kernel(x_src, src_idx, dst_idx, W, b):
    feat = jnp.pad(x_src.reshape(P, NSRC), ((0, 0), (0, NSRC_PAD - NSRC)))
    sidx = src_idx.reshape(P, ROWS, ROW)
    didx = dst_idx.reshape(P, ROWS, ROW)

    sc = pl.kernel(
        _sc_body,
        out_type=(jax.ShapeDtypeStruct((P * NDST_PAD,), jnp.float32),
                  jax.ShapeDtypeStruct((P * NDST_PAD,), jnp.float32)),
        mesh=plsc.VectorSubcoreMesh(core_axis_name="c", subcore_axis_name="s"),
        scratch_types=[
            pltpu.VMEM_SHARED((NSRC_PAD,), jnp.float32),
            pltpu.VMEM_SHARED((NSRC_PAD,), jnp.float32),
            pltpu.VMEM_SHARED((NDST_PAD,), jnp.float32),
            pltpu.VMEM_SHARED((NDST_PAD,), jnp.float32),
            pltpu.VMEM((RPT, ROW), jnp.int32),
            pltpu.VMEM((RPT, ROW), jnp.int32),
            pltpu.VMEM((NB, ROW), jnp.float32),
            pltpu.VMEM((ROW,), jnp.float32),
            pltpu.VMEM((SRC_CH,), jnp.float32),
            pltpu.VMEM((SRC_CH,), jnp.float32),
            pltpu.VMEM((SRC_CH,), jnp.float32),
            pltpu.SemaphoreType.DMA,
            pltpu.SemaphoreType.DMA,
            pltpu.SemaphoreType.DMA,
        ],
    )
    agg, degin = sc(feat.reshape(-1), sidx, didx)

    return pl.pallas_call(
        _tc_body,
        grid=(NDST // BN,),
        in_specs=[
            pl.BlockSpec((P, BN), lambda i: (0, i)),
            pl.BlockSpec((P, BN), lambda i: (0, i)),
            pl.BlockSpec((P, H), lambda i: (0, 0)),
            pl.BlockSpec((P, H), lambda i: (0, 0)),
        ],
        out_specs=pl.BlockSpec((BN, V, T, H), lambda i: (i, 0, 0, 0)),
        out_shape=jax.ShapeDtypeStruct((NDST, V, T, H), jnp.float32),
    )(agg.reshape(P, NDST_PAD), degin.reshape(P, NDST_PAD),
      W.reshape(P, H), b.reshape(P, H))


# speedup vs baseline: 246.5977x; 1.0146x over previous
import jax
import jax.numpy as jnp
from jax import lax
from jax.experimental import pallas as pl
from jax.experimental.pallas import tpu as pltpu
from jax.experimental.pallas import tpu_sc as plsc

V, T, NSRC, NDST, E, H = 2, 4, 50000, 12288, 800000, 128
P = V * T
NC, NS, L = 2, 16, 16
ROW = 128
ROWS = E // ROW
RPT = 392
NB = 56
NBLK = RPT // NB
TAILB = 6
TAILR = ROWS - 15 * RPT - TAILB * NB
NSRC_PAD = 50176
SRC_CH = NSRC_PAD // NS
NDST_PAD = 12416
DST_CH = NDST_PAD // NS
PPC = P // NC
BN = 512


def _sc_body(feat_h, sidx_h, didx_h, agg_h, degin_h,
             degout_sp, val_sp, agg_sp, degin_sp,
             sidx_v, didx_v, eval_v, ones_v, fbuf, dbuf, zbuf,
             sem0, sem1, sem2):
    c = lax.axis_index("c")
    w = lax.axis_index("s")

    @pl.loop(0, SRC_CH // L)
    def _init_zeros(i):
        zbuf[pl.ds(i * L, L)] = jnp.zeros((L,), jnp.float32)

    @pl.loop(0, ROW // L)
    def _init_ones(i):
        ones_v[pl.ds(i * L, L)] = jnp.ones((L,), jnp.float32)

    @pl.loop(0, PPC)
    def _pair(k):
        p = c * PPC + k

        pltpu.sync_copy(zbuf, degout_sp.at[pl.ds(w * SRC_CH, SRC_CH)])
        pltpu.sync_copy(zbuf.at[pl.ds(0, DST_CH)],
                        agg_sp.at[pl.ds(w * DST_CH, DST_CH)])
        pltpu.sync_copy(zbuf.at[pl.ds(0, DST_CH)],
                        degin_sp.at[pl.ds(w * DST_CH, DST_CH)])
        plsc.subcore_barrier()

        def _irow(buf, r):
            return buf.at[pl.ds(pl.multiple_of(r * ROW, ROW), ROW)]

        def _hist_block(row0, lrow0, nrows):
            e0 = pl.multiple_of(p * E + row0 * ROW, 8)
            pltpu.sync_copy(sidx_h.at[pl.ds(e0, nrows * ROW)],
                            sidx_v.at[pl.ds(lrow0 * ROW, nrows * ROW)])
            dd = pltpu.async_copy(didx_h.at[pl.ds(e0, nrows * ROW)],
                                  didx_v.at[pl.ds(lrow0 * ROW, nrows * ROW)],
                                  sem1)
            hs = [pltpu.async_copy(ones_v,
                                   degout_sp.at[_irow(sidx_v, lrow0 + j)],
                                   sem0, add=True) for j in range(nrows)]
            for d in hs:
                d.wait()
            dd.wait()

        nblk = jnp.where(w < NS - 1, NBLK, TAILB)

        @pl.loop(0, nblk)
        def _blk1(bi):
            _hist_block(pl.multiple_of(w * RPT + bi * NB, 8), bi * NB, NB)

        @pl.when(w == NS - 1)
        def _tail1():
            _hist_block((NS - 1) * RPT + TAILB * NB, TAILB * NB, TAILR)

        plsc.subcore_barrier()

        base = w * SRC_CH
        fbase = pl.multiple_of(p * NSRC_PAD + base, 8)
        pltpu.sync_copy(feat_h.at[pl.ds(fbase, SRC_CH)], fbuf)
        pltpu.sync_copy(degout_sp.at[pl.ds(base, SRC_CH)], dbuf)

        @pl.loop(0, SRC_CH // L)
        def _vec(i):
            off = i * L
            dg = jnp.maximum(dbuf[pl.ds(off, L)], 1.0)
            bits = lax.bitcast_convert_type(dg, jnp.int32)
            bits = jnp.int32(0x5F3759DF) - (bits >> 1)
            y = lax.bitcast_convert_type(bits, jnp.float32)
            hx = dg * 0.5
            y = y * (1.5 - hx * y * y)
            y = y * (1.5 - hx * y * y)
            y = y * (1.5 - hx * y * y)
            ft = fbuf[pl.ds(off, L)]
            ft = jnp.where(ft != ft, jnp.float32(0.0), ft)
            fbuf[pl.ds(off, L)] = ft * y

        pltpu.sync_copy(fbuf, val_sp.at[pl.ds(base, SRC_CH)])
        plsc.subcore_barrier()

        def _agg_block(lrow0, nrows):
            gs = [pltpu.async_copy(val_sp.at[_irow(sidx_v, lrow0 + j)],
                                   eval_v.at[j], sem0)
                  for j in range(nrows)]
            hs = [pltpu.async_copy(ones_v,
                                   degin_sp.at[_irow(didx_v, lrow0 + j)],
                                   sem1, add=True) for j in range(nrows)]
            for d in gs:
                d.wait()
            ss = [pltpu.async_copy(eval_v.at[j],
                                   agg_sp.at[_irow(didx_v, lrow0 + j)],
                                   sem2, add=True) for j in range(nrows)]
            for d in hs:
                d.wait()
            for d in ss:
                d.wait()

        @pl.loop(0, nblk)
        def _blk3(bi):
            _agg_block(bi * NB, NB)

        @pl.when(w == NS - 1)
        def _tail3():
            _agg_block(TAILB * NB, TAILR)

        plsc.subcore_barrier()

        off = w * DST_CH
        hoff = pl.multiple_of(p * NDST_PAD + off, 8)
        pltpu.sync_copy(agg_sp.at[pl.ds(off, DST_CH)],
                        fbuf.at[pl.ds(0, DST_CH)])
        pltpu.sync_copy(fbuf.at[pl.ds(0, DST_CH)],
                        agg_h.at[pl.ds(hoff, DST_CH)])
        pltpu.sync_copy(degin_sp.at[pl.ds(off, DST_CH)],
                        dbuf.at[pl.ds(0, DST_CH)])
        pltpu.sync_copy(dbuf.at[pl.ds(0, DST_CH)],
                        degin_h.at[pl.ds(hoff, DST_CH)])


def _tc_body(agg_r, deg_r, w_r, b_r, o_r):
    zall = (agg_r[...] * lax.rsqrt(jnp.maximum(deg_r[...], 1.0))).T
    for p in range(P):
        x = zall[:, p:p + 1] * w_r[p:p + 1, :] + b_r[p:p + 1, :]
        o_r[:, p // T, p % T, :] = jnp.where(x >= 0, x, 0.01 * x)


def kernel(x_src, src_idx, dst_idx, W, b):
    feat = jnp.pad(x_src.reshape(P, NSRC), ((0, 0), (0, NSRC_PAD - NSRC)))
    sidx = src_idx.reshape(-1)
    didx = dst_idx.reshape(-1)

    sc = pl.kernel(
        _sc_body,
        out_type=(jax.ShapeDtypeStruct((P * NDST_PAD,), jnp.float32),
                  jax.ShapeDtypeStruct((P * NDST_PAD,), jnp.float32)),
        mesh=plsc.VectorSubcoreMesh(core_axis_name="c", subcore_axis_name="s"),
        scratch_types=[
            pltpu.VMEM_SHARED((NSRC_PAD,), jnp.float32),
            pltpu.VMEM_SHARED((NSRC_PAD,), jnp.float32),
            pltpu.VMEM_SHARED((NDST_PAD,), jnp.float32),
            pltpu.VMEM_SHARED((NDST_PAD,), jnp.float32),
            pltpu.VMEM((RPT * ROW,), jnp.int32),
            pltpu.VMEM((RPT * ROW,), jnp.int32),
            pltpu.VMEM((NB, ROW), jnp.float32),
            pltpu.VMEM((ROW,), jnp.float32),
            pltpu.VMEM((SRC_CH,), jnp.float32),
            pltpu.VMEM((SRC_CH,), jnp.float32),
            pltpu.VMEM((SRC_CH,), jnp.float32),
            pltpu.SemaphoreType.DMA,
            pltpu.SemaphoreType.DMA,
            pltpu.SemaphoreType.DMA,
        ],
    )
    agg, degin = sc(feat.reshape(-1), sidx, didx)

    return pl.pallas_call(
        _tc_body,
        grid=(NDST // BN,),
        in_specs=[
            pl.BlockSpec((P, BN), lambda i: (0, i)),
            pl.BlockSpec((P, BN), lambda i: (0, i)),
            pl.BlockSpec((P, H), lambda i: (0, 0)),
            pl.BlockSpec((P, H), lambda i: (0, 0)),
        ],
        out_specs=pl.BlockSpec((BN, V, T, H), lambda i: (i, 0, 0, 0)),
        out_shape=jax.ShapeDtypeStruct((NDST, V, T, H), jnp.float32),
    )(agg.reshape(P, NDST_PAD), degin.reshape(P, NDST_PAD),
      W.reshape(P, H), b.reshape(P, H))
